# baseline (device time: 20942 ns/iter reference)
import jax
import jax.numpy as jnp
from jax import lax
from jax.experimental import pallas as pl
from jax.experimental.pallas import tpu as pltpu

N_DEV = 8
N_ROUNDS = 3
MASKS = (1, 3, 4)
MASK_ORDERS = ((1, 3, 4), (3, 4, 1))
B = 2
SQ = 128
H_LOC = 4
DH = 64
D_MODEL = 512
D_HEADS = H_LOC * DH


def kernel(x, Wq, K_ext, V_ext, Wo):
    my_pos = lax.axis_index("i")
    wq16 = lax.dynamic_slice(
        Wq, (0, my_pos * D_HEADS), (D_MODEL, D_HEADS)).astype(jnp.bfloat16)
    wo16 = lax.dynamic_slice(
        Wo, (my_pos * D_HEADS, 0), (D_HEADS, D_MODEL)).astype(jnp.bfloat16)

    def body(x_ref, wq_ref, k_ref, v_ref, wo_ref, out_ref,
             sbuf, comm, send_sems, recv_sems):
        my = lax.axis_index("i")

        barrier_sem = pltpu.get_barrier_semaphore()
        for m in MASKS:
            pl.semaphore_signal(
                barrier_sem, inc=1,
                device_id=(my ^ m,),
                device_id_type=pl.DeviceIdType.MESH,
            )
        pl.semaphore_wait(barrier_sem, N_ROUNDS)

        wq_s = wq_ref[...]
        wo_s = wo_ref[...]

        x_all = x_ref[...].reshape(B * SQ, D_MODEL).astype(jnp.bfloat16)
        q_all = jnp.dot(x_all, wq_s, preferred_element_type=jnp.float32)

        def compute_partial(b):
            q = q_all[b * SQ:(b + 1) * SQ]
            ctx_cols = []
            for h in range(H_LOC):
                qh = q[:, h * DH:(h + 1) * DH].astype(jnp.bfloat16)
                kh = k_ref[b, :, h, :].astype(jnp.bfloat16)
                vh = v_ref[b, :, h, :].astype(jnp.bfloat16)
                s = jnp.dot(qh, kh.T, preferred_element_type=jnp.float32) * 0.125
                w = jnp.exp(s)
                denom = jnp.sum(w, axis=-1, keepdims=True)
                ctx = jnp.dot(w.astype(jnp.bfloat16), vh,
                              preferred_element_type=jnp.float32) / denom
                ctx_cols.append(ctx.astype(jnp.bfloat16))
            ctx_all = jnp.concatenate(ctx_cols, axis=1)
            return jnp.dot(ctx_all, wo_s,
                           preferred_element_type=jnp.float32)

        def start_exchange(r, b):
            rdma = pltpu.make_async_remote_copy(
                src_ref=sbuf.at[b],
                dst_ref=comm.at[r, b],
                send_sem=send_sems.at[r, b],
                recv_sem=recv_sems.at[r, b],
                device_id=(my ^ MASK_ORDERS[b][r],),
                device_id_type=pl.DeviceIdType.MESH,
            )
            rdma.start()
            return rdma

        rdmas = {}
        for b in range(B):
            p = compute_partial(b)
            out_ref[b] = p
            sbuf[b] = p.astype(jnp.bfloat16)
            rdmas[(0, b)] = start_exchange(0, b)

        for r in range(N_ROUNDS):
            for b in range(B):
                rdmas[(r, b)].wait()
                new = out_ref[b] + comm[r, b].astype(jnp.float32)
                out_ref[b] = new
                if r + 1 < N_ROUNDS:
                    sbuf[b] = new.astype(jnp.bfloat16)
                    rdmas[(r + 1, b)] = start_exchange(r + 1, b)

    out_shape = jax.ShapeDtypeStruct((B, SQ, D_MODEL), jnp.float32)
    call = pl.pallas_call(
        body,
        out_shape=out_shape,
        in_specs=[pl.BlockSpec(memory_space=pltpu.VMEM)] * 5,
        out_specs=pl.BlockSpec(memory_space=pltpu.VMEM),
        scratch_shapes=[
            pltpu.VMEM((B, SQ, D_MODEL), jnp.bfloat16),
            pltpu.VMEM((N_ROUNDS, B, SQ, D_MODEL), jnp.bfloat16),
            pltpu.SemaphoreType.DMA((N_ROUNDS, B)),
            pltpu.SemaphoreType.DMA((N_ROUNDS, B)),
        ],
        compiler_params=pltpu.CompilerParams(collective_id=0),
    )
    return call(x, wq16, K_ext, V_ext, wo16)


# device time: 20618 ns/iter; 1.0157x vs baseline; 1.0157x over previous
import jax
import jax.numpy as jnp
from jax import lax
from jax.experimental import pallas as pl
from jax.experimental.pallas import tpu as pltpu

N_DEV = 8
N_ROUNDS = 3
MASKS = (1, 3, 4)
MASK_ORDERS = ((1, 3, 4), (3, 4, 1))
B = 2
SQ = 128
H_LOC = 4
DH = 64
D_MODEL = 512
D_HEADS = H_LOC * DH


def kernel(x, Wq, K_ext, V_ext, Wo):
    my_pos = lax.axis_index("i")
    wq16 = lax.dynamic_slice(
        Wq, (0, my_pos * D_HEADS), (D_MODEL, D_HEADS)).astype(jnp.bfloat16)
    wo16 = lax.dynamic_slice(
        Wo, (my_pos * D_HEADS, 0), (D_HEADS, D_MODEL)).astype(jnp.bfloat16)
    k16 = K_ext.astype(jnp.bfloat16).reshape(B, SQ, D_HEADS)
    v16 = V_ext.astype(jnp.bfloat16).reshape(B, SQ, D_HEADS)
    x16 = x.astype(jnp.bfloat16)

    def body(x_ref, wq_ref, k_ref, v_ref, wo_ref, out_ref,
             sbuf, comm, send_sems, recv_sems):
        my = lax.axis_index("i")

        barrier_sem = pltpu.get_barrier_semaphore()
        for m in MASKS:
            pl.semaphore_signal(
                barrier_sem, inc=1,
                device_id=(my ^ m,),
                device_id_type=pl.DeviceIdType.MESH,
            )
        pl.semaphore_wait(barrier_sem, N_ROUNDS)

        wq_s = wq_ref[...]
        wo_s = wo_ref[...]

        x_all = x_ref[...].reshape(B * SQ, D_MODEL)
        q_all = jnp.dot(x_all, wq_s, preferred_element_type=jnp.float32)

        def compute_partial(b):
            q = q_all[b * SQ:(b + 1) * SQ]
            ctx_cols = []
            for h in range(H_LOC):
                qh = q[:, h * DH:(h + 1) * DH].astype(jnp.bfloat16)
                kh = k_ref[b, :, h * DH:(h + 1) * DH]
                vh = v_ref[b, :, h * DH:(h + 1) * DH]
                s = jnp.dot(qh, kh.T, preferred_element_type=jnp.float32) * 0.125
                w = jnp.exp(s)
                denom = jnp.sum(w, axis=-1, keepdims=True)
                ctx = jnp.dot(w.astype(jnp.bfloat16), vh,
                              preferred_element_type=jnp.float32) / denom
                ctx_cols.append(ctx.astype(jnp.bfloat16))
            ctx_all = jnp.concatenate(ctx_cols, axis=1)
            return jnp.dot(ctx_all, wo_s,
                           preferred_element_type=jnp.float32)

        def start_exchange(r, b):
            rdma = pltpu.make_async_remote_copy(
                src_ref=sbuf.at[b],
                dst_ref=comm.at[r, b],
                send_sem=send_sems.at[r, b],
                recv_sem=recv_sems.at[r, b],
                device_id=(my ^ MASK_ORDERS[b][r],),
                device_id_type=pl.DeviceIdType.MESH,
            )
            rdma.start()
            return rdma

        rdmas = {}
        for b in range(B):
            p = compute_partial(b)
            sbuf[b] = p.astype(jnp.bfloat16)
            rdmas[(0, b)] = start_exchange(0, b)

        for r in range(N_ROUNDS):
            for b in range(B):
                rdmas[(r, b)].wait()
                sbuf[b] = sbuf[b] + comm[r, b]
                if r + 1 < N_ROUNDS:
                    rdmas[(r + 1, b)] = start_exchange(r + 1, b)
        out_ref[...] = sbuf[...].astype(jnp.float32)

    out_shape = jax.ShapeDtypeStruct((B, SQ, D_MODEL), jnp.float32)
    call = pl.pallas_call(
        body,
        out_shape=out_shape,
        in_specs=[pl.BlockSpec(memory_space=pltpu.VMEM)] * 5,
        out_specs=pl.BlockSpec(memory_space=pltpu.VMEM),
        scratch_shapes=[
            pltpu.VMEM((B, SQ, D_MODEL), jnp.bfloat16),
            pltpu.VMEM((N_ROUNDS, B, SQ, D_MODEL), jnp.bfloat16),
            pltpu.SemaphoreType.DMA((N_ROUNDS, B)),
            pltpu.SemaphoreType.DMA((N_ROUNDS, B)),
        ],
        compiler_params=pltpu.CompilerParams(collective_id=0),
    )
    return call(x16, wq16, k16, v16, wo16)


# device time: 20597 ns/iter; 1.0168x vs baseline; 1.0010x over previous
import jax
import jax.numpy as jnp
from jax import lax
from jax.experimental import pallas as pl
from jax.experimental.pallas import tpu as pltpu

N_DEV = 8
N_ROUNDS = 3
MASKS = (1, 3, 4)
MASK_ORDERS = ((1, 3, 4), (3, 4, 1))
B = 2
SQ = 128
H_LOC = 4
DH = 64
D_MODEL = 512
D_HEADS = H_LOC * DH


def kernel(x, Wq, K_ext, V_ext, Wo):
    my_pos = lax.axis_index("i")
    wq16 = lax.dynamic_slice(
        Wq, (0, my_pos * D_HEADS), (D_MODEL, D_HEADS)).astype(jnp.bfloat16)
    wo16 = lax.dynamic_slice(
        Wo, (my_pos * D_HEADS, 0), (D_HEADS, D_MODEL)).astype(jnp.bfloat16)
    k16 = K_ext.astype(jnp.bfloat16).reshape(B, SQ, D_HEADS)
    v16 = V_ext.astype(jnp.bfloat16).reshape(B, SQ, D_HEADS)
    x16 = x.astype(jnp.bfloat16)

    def body(x_ref, wq_ref, k_ref, v_ref, wo_ref, out_ref,
             sbuf, comm, send_sems, recv_sems):
        my = lax.axis_index("i")

        barrier_sem = pltpu.get_barrier_semaphore()
        for m in MASKS:
            pl.semaphore_signal(
                barrier_sem, inc=1,
                device_id=(my ^ m,),
                device_id_type=pl.DeviceIdType.MESH,
            )
        pl.semaphore_wait(barrier_sem, N_ROUNDS)

        wq_s = wq_ref[...]
        wo_s = wo_ref[...]

        x_all = x_ref[...].reshape(B * SQ, D_MODEL)
        q_all = jnp.dot(x_all, wq_s, preferred_element_type=jnp.float32)

        def compute_partial(b):
            q = q_all[b * SQ:(b + 1) * SQ]
            ctx_cols = []
            for h in range(H_LOC):
                qh = q[:, h * DH:(h + 1) * DH].astype(jnp.bfloat16)
                kh = k_ref[b, :, h * DH:(h + 1) * DH]
                vh = v_ref[b, :, h * DH:(h + 1) * DH]
                s = jnp.dot(qh, kh.T, preferred_element_type=jnp.float32) * 0.125
                w = jnp.exp(s)
                denom = jnp.sum(w, axis=-1, keepdims=True)
                ctx = jnp.dot(w.astype(jnp.bfloat16), vh,
                              preferred_element_type=jnp.float32) / denom
                ctx_cols.append(ctx.astype(jnp.bfloat16))
            ctx_all = jnp.concatenate(ctx_cols, axis=1)
            return jnp.dot(ctx_all, wo_s,
                           preferred_element_type=jnp.float32)

        def start_exchange(r, b):
            rdma = pltpu.make_async_remote_copy(
                src_ref=sbuf.at[b],
                dst_ref=comm.at[r, b],
                send_sem=send_sems.at[r, b],
                recv_sem=recv_sems.at[r, b],
                device_id=(my ^ MASK_ORDERS[b][r],),
                device_id_type=pl.DeviceIdType.MESH,
            )
            rdma.start()
            return rdma

        rdmas = {}
        for b in range(B):
            p = compute_partial(b)
            out_ref[b] = p
            sbuf[b] = p.astype(jnp.bfloat16)
            rdmas[(0, b)] = start_exchange(0, b)

        for r in range(N_ROUNDS):
            for b in range(B):
                rdmas[(r, b)].wait()
                new = out_ref[b] + comm[r, b].astype(jnp.float32)
                out_ref[b] = new
                if r + 1 < N_ROUNDS:
                    sbuf[b] = new.astype(jnp.bfloat16)
                    rdmas[(r + 1, b)] = start_exchange(r + 1, b)

    out_shape = jax.ShapeDtypeStruct((B, SQ, D_MODEL), jnp.float32)
    call = pl.pallas_call(
        body,
        out_shape=out_shape,
        in_specs=[pl.BlockSpec(memory_space=pltpu.VMEM)] * 5,
        out_specs=pl.BlockSpec(memory_space=pltpu.VMEM),
        scratch_shapes=[
            pltpu.VMEM((B, SQ, D_MODEL), jnp.bfloat16),
            pltpu.VMEM((N_ROUNDS, B, SQ, D_MODEL), jnp.bfloat16),
            pltpu.SemaphoreType.DMA((N_ROUNDS, B)),
            pltpu.SemaphoreType.DMA((N_ROUNDS, B)),
        ],
        compiler_params=pltpu.CompilerParams(collective_id=0),
    )
    return call(x16, wq16, k16, v16, wo16)


# device time: 20548 ns/iter; 1.0192x vs baseline; 1.0024x over previous
import jax
import jax.numpy as jnp
from jax import lax
from jax.experimental import pallas as pl
from jax.experimental.pallas import tpu as pltpu

N_DEV = 8
N_ROUNDS = 3
MASKS = (1, 3, 4)
MASK_ORDERS = ((1, 3, 4), (3, 4, 1))
B = 2
SQ = 128
H_LOC = 4
DH = 64
D_MODEL = 512
D_HEADS = H_LOC * DH


def kernel(x, Wq, K_ext, V_ext, Wo):
    my_pos = lax.axis_index("i")
    wq16 = lax.dynamic_slice(
        Wq, (0, my_pos * D_HEADS), (D_MODEL, D_HEADS)).astype(jnp.bfloat16)
    wo16 = lax.dynamic_slice(
        Wo, (my_pos * D_HEADS, 0), (D_HEADS, D_MODEL)).astype(jnp.bfloat16)
    k16 = K_ext.astype(jnp.bfloat16).reshape(B, SQ, D_HEADS)
    v16 = V_ext.astype(jnp.bfloat16).reshape(B, SQ, D_HEADS)
    x16 = x

    def body(x_ref, wq_ref, k_ref, v_ref, wo_ref, out_ref,
             sbuf, comm, send_sems, recv_sems):
        my = lax.axis_index("i")

        barrier_sem = pltpu.get_barrier_semaphore()
        for m in MASKS:
            pl.semaphore_signal(
                barrier_sem, inc=1,
                device_id=(my ^ m,),
                device_id_type=pl.DeviceIdType.MESH,
            )
        pl.semaphore_wait(barrier_sem, N_ROUNDS)

        wq_s = wq_ref[...]
        wo_s = wo_ref[...]

        x_all = x_ref[...].reshape(B * SQ, D_MODEL).astype(jnp.bfloat16)
        q_all = jnp.dot(x_all, wq_s, preferred_element_type=jnp.float32)

        def compute_partial(b):
            q = q_all[b * SQ:(b + 1) * SQ]
            ctx_cols = []
            for h in range(H_LOC):
                qh = q[:, h * DH:(h + 1) * DH].astype(jnp.bfloat16)
                kh = k_ref[b, :, h * DH:(h + 1) * DH]
                vh = v_ref[b, :, h * DH:(h + 1) * DH]
                s = jnp.dot(qh, kh.T, preferred_element_type=jnp.float32) * 0.125
                w = jnp.exp(s)
                denom = jnp.sum(w, axis=-1, keepdims=True)
                ctx = jnp.dot(w.astype(jnp.bfloat16), vh,
                              preferred_element_type=jnp.float32) / denom
                ctx_cols.append(ctx.astype(jnp.bfloat16))
            ctx_all = jnp.concatenate(ctx_cols, axis=1)
            return jnp.dot(ctx_all, wo_s,
                           preferred_element_type=jnp.float32)

        def start_exchange(r, b):
            rdma = pltpu.make_async_remote_copy(
                src_ref=sbuf.at[b],
                dst_ref=comm.at[r, b],
                send_sem=send_sems.at[r, b],
                recv_sem=recv_sems.at[r, b],
                device_id=(my ^ MASK_ORDERS[b][r],),
                device_id_type=pl.DeviceIdType.MESH,
            )
            rdma.start()
            return rdma

        rdmas = {}
        for b in range(B):
            p = compute_partial(b)
            out_ref[b] = p
            sbuf[b] = p.astype(jnp.bfloat16)
            rdmas[(0, b)] = start_exchange(0, b)

        for r in range(N_ROUNDS):
            for b in range(B):
                rdmas[(r, b)].wait()
                new = out_ref[b] + comm[r, b].astype(jnp.float32)
                out_ref[b] = new
                if r + 1 < N_ROUNDS:
                    sbuf[b] = new.astype(jnp.bfloat16)
                    rdmas[(r + 1, b)] = start_exchange(r + 1, b)

    out_shape = jax.ShapeDtypeStruct((B, SQ, D_MODEL), jnp.float32)
    call = pl.pallas_call(
        body,
        out_shape=out_shape,
        in_specs=[pl.BlockSpec(memory_space=pltpu.VMEM)] * 5,
        out_specs=pl.BlockSpec(memory_space=pltpu.VMEM),
        scratch_shapes=[
            pltpu.VMEM((B, SQ, D_MODEL), jnp.bfloat16),
            pltpu.VMEM((N_ROUNDS, B, SQ, D_MODEL), jnp.bfloat16),
            pltpu.SemaphoreType.DMA((N_ROUNDS, B)),
            pltpu.SemaphoreType.DMA((N_ROUNDS, B)),
        ],
        compiler_params=pltpu.CompilerParams(collective_id=0),
    )
    return call(x16, wq16, k16, v16, wo16)


# device time: 18522 ns/iter; 1.1307x vs baseline; 1.1094x over previous
import jax
import jax.numpy as jnp
from jax import lax
from jax.experimental import pallas as pl
from jax.experimental.pallas import tpu as pltpu

N_DEV = 8
N_ROUNDS = 3
MASKS = (1, 3, 4)
MASK_ORDERS = ((1, 3, 4), (3, 4, 1), (4, 1, 3), (1, 3, 4))
N_SLICES = 4
SQ_H = 64
B = 2
SQ = 128
H_LOC = 4
DH = 64
D_MODEL = 512
D_HEADS = H_LOC * DH


def kernel(x, Wq, K_ext, V_ext, Wo):
    my_pos = lax.axis_index("i")
    wq16 = lax.dynamic_slice(
        Wq, (0, my_pos * D_HEADS), (D_MODEL, D_HEADS)).astype(jnp.bfloat16)
    wo16 = lax.dynamic_slice(
        Wo, (my_pos * D_HEADS, 0), (D_HEADS, D_MODEL)).astype(jnp.bfloat16)
    k16 = K_ext.astype(jnp.bfloat16).reshape(B, SQ, D_HEADS)
    v16 = V_ext.astype(jnp.bfloat16).reshape(B, SQ, D_HEADS)
    x16 = x

    def body(x_ref, wq_ref, k_ref, v_ref, wo_ref, out_ref,
             sbuf, comm, send_sems, recv_sems):
        my = lax.axis_index("i")

        barrier_sem = pltpu.get_barrier_semaphore()
        for m in MASKS:
            pl.semaphore_signal(
                barrier_sem, inc=1,
                device_id=(my ^ m,),
                device_id_type=pl.DeviceIdType.MESH,
            )
        pl.semaphore_wait(barrier_sem, N_ROUNDS)

        wq_s = wq_ref[...]
        wo_s = wo_ref[...]

        x_all = x_ref[...].reshape(B * SQ, D_MODEL).astype(jnp.bfloat16)
        q_all = jnp.dot(x_all, wq_s, preferred_element_type=jnp.float32)

        def compute_partial(b):
            q = q_all[b * SQ:(b + 1) * SQ]
            ctx_cols = []
            for h in range(H_LOC):
                qh = q[:, h * DH:(h + 1) * DH].astype(jnp.bfloat16)
                kh = k_ref[b, :, h * DH:(h + 1) * DH]
                vh = v_ref[b, :, h * DH:(h + 1) * DH]
                s = jnp.dot(qh, kh.T, preferred_element_type=jnp.float32) * 0.125
                w = jnp.exp(s)
                denom = jnp.sum(w, axis=-1, keepdims=True)
                ctx = jnp.dot(w.astype(jnp.bfloat16), vh,
                              preferred_element_type=jnp.float32) / denom
                ctx_cols.append(ctx.astype(jnp.bfloat16))
            ctx_all = jnp.concatenate(ctx_cols, axis=1)
            return jnp.dot(ctx_all, wo_s,
                           preferred_element_type=jnp.float32)

        def start_exchange(r, s):
            rdma = pltpu.make_async_remote_copy(
                src_ref=sbuf.at[s],
                dst_ref=comm.at[r, s],
                send_sem=send_sems.at[r, s],
                recv_sem=recv_sems.at[r, s],
                device_id=(my ^ MASK_ORDERS[s][r],),
                device_id_type=pl.DeviceIdType.MESH,
            )
            rdma.start()
            return rdma

        rdmas = {}
        for b in range(B):
            p = compute_partial(b)
            out_ref[b] = p
            for h2 in range(2):
                s = 2 * b + h2
                sbuf[s] = p[h2 * SQ_H:(h2 + 1) * SQ_H].astype(jnp.bfloat16)
                rdmas[(0, s)] = start_exchange(0, s)

        for r in range(N_ROUNDS):
            for s in range(N_SLICES):
                b, h2 = s // 2, s % 2
                rdmas[(r, s)].wait()
                new = (out_ref[b, h2 * SQ_H:(h2 + 1) * SQ_H]
                       + comm[r, s].astype(jnp.float32))
                out_ref[b, h2 * SQ_H:(h2 + 1) * SQ_H] = new
                if r + 1 < N_ROUNDS:
                    sbuf[s] = new.astype(jnp.bfloat16)
                    rdmas[(r + 1, s)] = start_exchange(r + 1, s)

    out_shape = jax.ShapeDtypeStruct((B, SQ, D_MODEL), jnp.float32)
    call = pl.pallas_call(
        body,
        out_shape=out_shape,
        in_specs=[pl.BlockSpec(memory_space=pltpu.VMEM)] * 5,
        out_specs=pl.BlockSpec(memory_space=pltpu.VMEM),
        scratch_shapes=[
            pltpu.VMEM((N_SLICES, SQ_H, D_MODEL), jnp.bfloat16),
            pltpu.VMEM((N_ROUNDS, N_SLICES, SQ_H, D_MODEL), jnp.bfloat16),
            pltpu.SemaphoreType.DMA((N_ROUNDS, N_SLICES)),
            pltpu.SemaphoreType.DMA((N_ROUNDS, N_SLICES)),
        ],
        compiler_params=pltpu.CompilerParams(collective_id=0),
    )
    return call(x16, wq16, k16, v16, wo16)


# device time: 18005 ns/iter; 1.1631x vs baseline; 1.0287x over previous
import jax
import jax.numpy as jnp
from jax import lax
from jax.experimental import pallas as pl
from jax.experimental.pallas import tpu as pltpu

N_DEV = 8
N_ROUNDS = 3
MASKS = (1, 3, 4)
MASK_ORDERS = ((1, 3, 4), (3, 4, 1), (4, 1, 3), (1, 3, 4),
               (3, 4, 1), (4, 1, 3), (1, 3, 4), (3, 4, 1))
N_SLICES = 8
SLICES_PER_B = 4
SQ_H = 32
B = 2
SQ = 128
H_LOC = 4
DH = 64
D_MODEL = 512
D_HEADS = H_LOC * DH


def kernel(x, Wq, K_ext, V_ext, Wo):
    my_pos = lax.axis_index("i")
    wq16 = lax.dynamic_slice(
        Wq, (0, my_pos * D_HEADS), (D_MODEL, D_HEADS)).astype(jnp.bfloat16)
    wo16 = lax.dynamic_slice(
        Wo, (my_pos * D_HEADS, 0), (D_HEADS, D_MODEL)).astype(jnp.bfloat16)
    k16 = K_ext.astype(jnp.bfloat16).reshape(B, SQ, D_HEADS)
    v16 = V_ext.astype(jnp.bfloat16).reshape(B, SQ, D_HEADS)
    x16 = x

    def body(x_ref, wq_ref, k_ref, v_ref, wo_ref, out_ref,
             sbuf, comm, send_sems, recv_sems):
        my = lax.axis_index("i")

        barrier_sem = pltpu.get_barrier_semaphore()
        for m in MASKS:
            pl.semaphore_signal(
                barrier_sem, inc=1,
                device_id=(my ^ m,),
                device_id_type=pl.DeviceIdType.MESH,
            )
        pl.semaphore_wait(barrier_sem, N_ROUNDS)

        wq_s = wq_ref[...]
        wo_s = wo_ref[...]

        x_all = x_ref[...].reshape(B * SQ, D_MODEL).astype(jnp.bfloat16)
        q_all = jnp.dot(x_all, wq_s, preferred_element_type=jnp.float32)

        def compute_partial(b):
            q = q_all[b * SQ:(b + 1) * SQ]
            ctx_cols = []
            for h in range(H_LOC):
                qh = q[:, h * DH:(h + 1) * DH].astype(jnp.bfloat16)
                kh = k_ref[b, :, h * DH:(h + 1) * DH]
                vh = v_ref[b, :, h * DH:(h + 1) * DH]
                s = jnp.dot(qh, kh.T, preferred_element_type=jnp.float32) * 0.125
                w = jnp.exp(s)
                denom = jnp.sum(w, axis=-1, keepdims=True)
                ctx = jnp.dot(w.astype(jnp.bfloat16), vh,
                              preferred_element_type=jnp.float32) / denom
                ctx_cols.append(ctx.astype(jnp.bfloat16))
            ctx_all = jnp.concatenate(ctx_cols, axis=1)
            return jnp.dot(ctx_all, wo_s,
                           preferred_element_type=jnp.float32)

        def start_exchange(r, s):
            rdma = pltpu.make_async_remote_copy(
                src_ref=sbuf.at[s],
                dst_ref=comm.at[r, s],
                send_sem=send_sems.at[r, s],
                recv_sem=recv_sems.at[r, s],
                device_id=(my ^ MASK_ORDERS[s][r],),
                device_id_type=pl.DeviceIdType.MESH,
            )
            rdma.start()
            return rdma

        rdmas = {}
        for b in range(B):
            p = compute_partial(b)
            out_ref[b] = p
            for h2 in range(SLICES_PER_B):
                s = SLICES_PER_B * b + h2
                sbuf[s] = p[h2 * SQ_H:(h2 + 1) * SQ_H].astype(jnp.bfloat16)
                rdmas[(0, s)] = start_exchange(0, s)

        for r in range(N_ROUNDS):
            for s in range(N_SLICES):
                b, h2 = s // SLICES_PER_B, s % SLICES_PER_B
                rdmas[(r, s)].wait()
                new = (out_ref[b, h2 * SQ_H:(h2 + 1) * SQ_H]
                       + comm[r, s].astype(jnp.float32))
                out_ref[b, h2 * SQ_H:(h2 + 1) * SQ_H] = new
                if r + 1 < N_ROUNDS:
                    sbuf[s] = new.astype(jnp.bfloat16)
                    rdmas[(r + 1, s)] = start_exchange(r + 1, s)

    out_shape = jax.ShapeDtypeStruct((B, SQ, D_MODEL), jnp.float32)
    call = pl.pallas_call(
        body,
        out_shape=out_shape,
        in_specs=[pl.BlockSpec(memory_space=pltpu.VMEM)] * 5,
        out_specs=pl.BlockSpec(memory_space=pltpu.VMEM),
        scratch_shapes=[
            pltpu.VMEM((N_SLICES, SQ_H, D_MODEL), jnp.bfloat16),
            pltpu.VMEM((N_ROUNDS, N_SLICES, SQ_H, D_MODEL), jnp.bfloat16),
            pltpu.SemaphoreType.DMA((N_ROUNDS, N_SLICES)),
            pltpu.SemaphoreType.DMA((N_ROUNDS, N_SLICES)),
        ],
        compiler_params=pltpu.CompilerParams(collective_id=0),
    )
    return call(x16, wq16, k16, v16, wo16)


# device time: 17988 ns/iter; 1.1642x vs baseline; 1.0009x over previous
import jax
import jax.numpy as jnp
from jax import lax
from jax.experimental import pallas as pl
from jax.experimental.pallas import tpu as pltpu

N_DEV = 8
N_ROUNDS = 3
MASKS = (1, 3, 4)
_ROT = ((1, 3, 4), (3, 4, 1), (4, 1, 3))
N_SLICES = 16
SLICES_PER_B = 8
SQ_H = 16
MASK_ORDERS = tuple(_ROT[s % 3] for s in range(N_SLICES))
B = 2
SQ = 128
H_LOC = 4
DH = 64
D_MODEL = 512
D_HEADS = H_LOC * DH


def kernel(x, Wq, K_ext, V_ext, Wo):
    my_pos = lax.axis_index("i")
    wq16 = lax.dynamic_slice(
        Wq, (0, my_pos * D_HEADS), (D_MODEL, D_HEADS)).astype(jnp.bfloat16)
    wo16 = lax.dynamic_slice(
        Wo, (my_pos * D_HEADS, 0), (D_HEADS, D_MODEL)).astype(jnp.bfloat16)
    k16 = K_ext.astype(jnp.bfloat16).reshape(B, SQ, D_HEADS)
    v16 = V_ext.astype(jnp.bfloat16).reshape(B, SQ, D_HEADS)
    x16 = x

    def body(x_ref, wq_ref, k_ref, v_ref, wo_ref, out_ref,
             sbuf, comm, send_sems, recv_sems):
        my = lax.axis_index("i")

        barrier_sem = pltpu.get_barrier_semaphore()
        for m in MASKS:
            pl.semaphore_signal(
                barrier_sem, inc=1,
                device_id=(my ^ m,),
                device_id_type=pl.DeviceIdType.MESH,
            )
        pl.semaphore_wait(barrier_sem, N_ROUNDS)

        wq_s = wq_ref[...]
        wo_s = wo_ref[...]

        x_all = x_ref[...].reshape(B * SQ, D_MODEL).astype(jnp.bfloat16)
        q_all = jnp.dot(x_all, wq_s, preferred_element_type=jnp.float32)

        def compute_partial(b):
            q = q_all[b * SQ:(b + 1) * SQ]
            ctx_cols = []
            for h in range(H_LOC):
                qh = q[:, h * DH:(h + 1) * DH].astype(jnp.bfloat16)
                kh = k_ref[b, :, h * DH:(h + 1) * DH]
                vh = v_ref[b, :, h * DH:(h + 1) * DH]
                s = jnp.dot(qh, kh.T, preferred_element_type=jnp.float32) * 0.125
                w = jnp.exp(s)
                denom = jnp.sum(w, axis=-1, keepdims=True)
                ctx = jnp.dot(w.astype(jnp.bfloat16), vh,
                              preferred_element_type=jnp.float32) / denom
                ctx_cols.append(ctx.astype(jnp.bfloat16))
            ctx_all = jnp.concatenate(ctx_cols, axis=1)
            return jnp.dot(ctx_all, wo_s,
                           preferred_element_type=jnp.float32)

        def start_exchange(r, s):
            rdma = pltpu.make_async_remote_copy(
                src_ref=sbuf.at[s],
                dst_ref=comm.at[r, s],
                send_sem=send_sems.at[r, s],
                recv_sem=recv_sems.at[r, s],
                device_id=(my ^ MASK_ORDERS[s][r],),
                device_id_type=pl.DeviceIdType.MESH,
            )
            rdma.start()
            return rdma

        rdmas = {}
        for b in range(B):
            p = compute_partial(b)
            out_ref[b] = p
            for h2 in range(SLICES_PER_B):
                s = SLICES_PER_B * b + h2
                sbuf[s] = p[h2 * SQ_H:(h2 + 1) * SQ_H].astype(jnp.bfloat16)
                rdmas[(0, s)] = start_exchange(0, s)

        for r in range(N_ROUNDS):
            for s in range(N_SLICES):
                b, h2 = s // SLICES_PER_B, s % SLICES_PER_B
                rdmas[(r, s)].wait()
                new = (out_ref[b, h2 * SQ_H:(h2 + 1) * SQ_H]
                       + comm[r, s].astype(jnp.float32))
                out_ref[b, h2 * SQ_H:(h2 + 1) * SQ_H] = new
                if r + 1 < N_ROUNDS:
                    sbuf[s] = new.astype(jnp.bfloat16)
                    rdmas[(r + 1, s)] = start_exchange(r + 1, s)

    out_shape = jax.ShapeDtypeStruct((B, SQ, D_MODEL), jnp.float32)
    call = pl.pallas_call(
        body,
        out_shape=out_shape,
        in_specs=[pl.BlockSpec(memory_space=pltpu.VMEM)] * 5,
        out_specs=pl.BlockSpec(memory_space=pltpu.VMEM),
        scratch_shapes=[
            pltpu.VMEM((N_SLICES, SQ_H, D_MODEL), jnp.bfloat16),
            pltpu.VMEM((N_ROUNDS, N_SLICES, SQ_H, D_MODEL), jnp.bfloat16),
            pltpu.SemaphoreType.DMA((N_ROUNDS, N_SLICES)),
            pltpu.SemaphoreType.DMA((N_ROUNDS, N_SLICES)),
        ],
        compiler_params=pltpu.CompilerParams(collective_id=0),
    )
    return call(x16, wq16, k16, v16, wo16)
